# baseline (speedup 1.0000x reference)
import functools

import jax
import jax.numpy as jnp
from jax import lax
from jax.experimental import pallas as pl
from jax.experimental.pallas import tpu as pltpu
from jax.experimental.pallas import tpu_sc as plsc

MAZE = 32
P = MAZE * MAZE
D = 64
V = 4
PP = P // 2
TBL = V * V * PP
NC, NS, L = 2, 16, 16
NW = NC * NS
CH = 64
NBUF = 4


def _sc_encode(maze128, cellcat, pos128, batch):
    n_pairs = batch * PP
    per_w = n_pairs // NW
    nchunks = per_w // CH
    rows_per_sub = TBL // NS

    mesh = plsc.VectorSubcoreMesh(core_axis_name="c", subcore_axis_name="s")

    @functools.partial(
        pl.kernel,
        out_type=jax.ShapeDtypeStruct((n_pairs, 2 * D), jnp.float32),
        mesh=mesh,
        compiler_params=pltpu.CompilerParams(needs_layout_passes=False),
        scratch_types=[
            pltpu.VMEM_SHARED((TBL, 2 * D), jnp.float32),
            pltpu.VMEM((2 * D,), jnp.float32),
        ] + [pltpu.VMEM((2 * CH,), jnp.int32) for _ in range(NBUF)]
          + [pltpu.VMEM((CH,), jnp.int32) for _ in range(NBUF)]
          + [pltpu.VMEM((CH, 2 * D), jnp.float32) for _ in range(NBUF)]
          + [pltpu.SemaphoreType.DMA for _ in range(3 * NBUF)],
    )
    def k(maze_hbm, cell_hbm, pos_hbm, out_hbm, tbl_sh, ccbuf, *ring):
        mbufs = ring[:NBUF]
        ibufs = ring[NBUF:2 * NBUF]
        obufs = ring[2 * NBUF:3 * NBUF]
        msems = ring[3 * NBUF:4 * NBUF]
        gsems = ring[4 * NBUF:5 * NBUF]
        osems = ring[5 * NBUF:6 * NBUF]
        cid = lax.axis_index("c")
        sid = lax.axis_index("s")
        wid = sid * NC + cid

        pltpu.sync_copy(cell_hbm.at[sid], ccbuf)
        ccs = [ccbuf[pl.ds(j * L, L)] for j in range(2 * D // L)]
        row0 = sid * rows_per_sub
        for kk in range(rows_per_sub // CH):
            bb = obufs[kk % NBUF]
            pltpu.sync_copy(pos_hbm.at[pl.ds(kk * CH, CH)], bb)

            def add_row(r, _, bb=bb):
                for j in range(2 * D // L):
                    bb[r, pl.ds(j * L, L)] += ccs[j]
                return _

            lax.fori_loop(0, CH, add_row, 0)
            pltpu.sync_copy(bb, tbl_sh.at[pl.ds(row0 + kk * CH, CH)])
        plsc.subcore_barrier()

        pr0 = wid * per_w
        lanes = lax.iota(jnp.int32, L)

        def issue_maze(t, b):
            pltpu.async_copy(maze_hbm.at[(pr0 + t * CH) // 64], mbufs[b],
                             msems[b])

        def wait_maze(t, b):
            pltpu.make_async_copy(maze_hbm.at[(pr0 + t * CH) // 64], mbufs[b],
                                  msems[b]).wait()

        def issue_gather(t, b):
            ppb = lax.rem(t, PP // CH) * CH
            for j in range(CH // L):
                e = plsc.load_gather(mbufs[b], [32 * j + 2 * lanes])
                o = plsc.load_gather(mbufs[b], [32 * j + 2 * lanes + 1])
                ibufs[b][pl.ds(j * L, L)] = (
                    (e * V + o) * PP + (ppb + j * L) + lanes)
            pltpu.async_copy(tbl_sh.at[ibufs[b]], obufs[b], gsems[b])

        def wait_gather(b):
            pltpu.make_async_copy(tbl_sh.at[ibufs[b]], obufs[b], gsems[b]).wait()

        def issue_out(t, b):
            pltpu.async_copy(obufs[b], out_hbm.at[pl.ds(pr0 + t * CH, CH)],
                             osems[b])

        def wait_out(t, b):
            pltpu.make_async_copy(obufs[b],
                                  out_hbm.at[pl.ds(pr0 + t * CH, CH)],
                                  osems[b]).wait()

        for b in range(NBUF):
            issue_maze(b, b)
        for b in range(NBUF):
            wait_maze(b, b)
            issue_gather(b, b)
            issue_maze(b + NBUF, b)
            if b >= 1:
                wait_gather(b - 1)
                issue_out(b - 1, b - 1)

        def group(gi, _):
            for b in range(NBUF):
                t = gi * NBUF + b
                wait_out(t - NBUF, b)
                wait_maze(t, b)
                issue_gather(t, b)
                issue_maze(t + NBUF, b)
                prev = (b - 1) % NBUF
                wait_gather(prev)
                issue_out(t - 1, prev)
            return _

        lax.fori_loop(1, nchunks // NBUF - 1, group, 0)

        gi_last = nchunks // NBUF - 1
        for b in range(NBUF):
            t = gi_last * NBUF + b
            wait_out(t - NBUF, b)
            wait_maze(t, b)
            issue_gather(t, b)
            prev = (b - 1) % NBUF
            wait_gather(prev)
            issue_out(t - 1, prev)

        last = nchunks - 1
        wait_gather(last % NBUF)
        issue_out(last, last % NBUF)
        for b in range(NBUF):
            t = nchunks - NBUF + b
            wait_out(t, b)

    return k(maze128, cellcat, pos128)


def kernel(maze_grid, cell_table, pos_table):
    batch, h, w = maze_grid.shape
    maze128 = maze_grid.astype(jnp.int32).reshape(batch * h * w // 128, 128)
    cellcat = jnp.concatenate(
        [jnp.repeat(cell_table, V, axis=0),
         jnp.tile(cell_table, (V, 1))], axis=1)
    pos128 = pos_table.reshape(P // 2, 2 * D)
    out = _sc_encode(maze128, cellcat, pos128, batch)
    return out.reshape(batch, h * w, D)
